# Initial kernel scaffold; baseline (speedup 1.0000x reference)
import jax
import jax.numpy as jnp
from jax.experimental import pallas as pl


def kernel(X, W0, b0, W1, b1, v_idx, e_idx, num_hyperedges):
    raise NotImplementedError("write your pallas kernel here")


# speedup vs baseline: 7.0112x; 7.0112x over previous
import functools

import jax
import jax.numpy as jnp
from jax import lax
from jax.experimental import pallas as pl
from jax.experimental.pallas import tpu as pltpu
from jax.experimental.pallas import tpu_sc as plsc

N_NODES = 10000
N_INC = 320000
N_HE = 2500
N_HE_PAD = 2560
HE_SLICE = 160
N_NODES_PAD = 10112
NODE_SLICE = 632
NUM_CORES = 2
NUM_SUBCORES = 16
NW = NUM_CORES * NUM_SUBCORES
PER_TILE = N_INC // NW
CHUNK = 80
NCHUNK = PER_TILE // CHUNK

_MESH = plsc.VectorSubcoreMesh(
    core_axis_name="c", subcore_axis_name="s",
    num_cores=NUM_CORES, num_subcores=NUM_SUBCORES)


def _degrees_body(vi_hbm, ei_hbm, ones_hbm, z_hbm, dv_out, de_out,
                  vbuf, ebuf, ones_v, dv_sh, de_sh):
    cid = lax.axis_index("c")
    sid = lax.axis_index("s")
    wid = cid * NUM_SUBCORES + sid
    pltpu.sync_copy(z_hbm, dv_sh.at[pl.ds(sid * NODE_SLICE, NODE_SLICE)])
    pltpu.sync_copy(z_hbm.at[pl.ds(0, HE_SLICE)],
                    de_sh.at[pl.ds(sid * HE_SLICE, HE_SLICE)])
    pltpu.sync_copy(ones_hbm, ones_v)
    pltpu.sync_copy(vi_hbm.at[wid], vbuf)
    pltpu.sync_copy(ei_hbm.at[wid], ebuf)
    plsc.subcore_barrier()

    @pl.loop(0, NCHUNK)
    def _(j):
        pltpu.sync_copy(ones_v, dv_sh.at[vbuf.at[j]], add=True)
        pltpu.sync_copy(ones_v, de_sh.at[ebuf.at[j]], add=True)

    plsc.subcore_barrier()
    pltpu.sync_copy(dv_sh.at[pl.ds(sid * NODE_SLICE, NODE_SLICE)],
                    dv_out.at[cid, pl.ds(sid * NODE_SLICE, NODE_SLICE)])
    pltpu.sync_copy(de_sh.at[pl.ds(sid * HE_SLICE, HE_SLICE)],
                    de_out.at[cid, pl.ds(sid * HE_SLICE, HE_SLICE)])


def _make_degrees():
    return pl.kernel(
        _degrees_body,
        out_type=(jax.ShapeDtypeStruct((NUM_CORES, N_NODES_PAD, 16), jnp.float32),
                  jax.ShapeDtypeStruct((NUM_CORES, N_HE_PAD, 16), jnp.float32)),
        mesh=_MESH,
        scratch_types=[
            pltpu.VMEM((NCHUNK, CHUNK), jnp.int32),
            pltpu.VMEM((NCHUNK, CHUNK), jnp.int32),
            pltpu.VMEM((CHUNK, 16), jnp.float32),
            pltpu.VMEM_SHARED((N_NODES_PAD, 16), jnp.float32),
            pltpu.VMEM_SHARED((N_HE_PAD, 16), jnp.float32),
        ])


def _v2e_body(c, h_hbm, vi_hbm, ei_hbm, z_hbm, he_out,
              vbuf, ebuf, rows, he_sh):
    cid = lax.axis_index("c")
    sid = lax.axis_index("s")
    wid = cid * NUM_SUBCORES + sid
    pltpu.sync_copy(z_hbm, he_sh.at[pl.ds(sid * HE_SLICE, HE_SLICE)])
    pltpu.sync_copy(vi_hbm.at[wid], vbuf)
    pltpu.sync_copy(ei_hbm.at[wid], ebuf)
    plsc.subcore_barrier()

    @pl.loop(0, NCHUNK)
    def _(j):
        pltpu.sync_copy(h_hbm.at[vbuf.at[j]], rows)
        pltpu.sync_copy(rows, he_sh.at[ebuf.at[j]], add=True)

    plsc.subcore_barrier()
    pltpu.sync_copy(he_sh.at[pl.ds(sid * HE_SLICE, HE_SLICE)],
                    he_out.at[cid, pl.ds(sid * HE_SLICE, HE_SLICE)])


def _make_v2e(c):
    return pl.kernel(
        functools.partial(_v2e_body, c),
        out_type=jax.ShapeDtypeStruct((NUM_CORES, N_HE_PAD, c), jnp.float32),
        mesh=_MESH,
        scratch_types=[
            pltpu.VMEM((NCHUNK, CHUNK), jnp.int32),
            pltpu.VMEM((NCHUNK, CHUNK), jnp.int32),
            pltpu.VMEM((CHUNK, c), jnp.float32),
            pltpu.VMEM_SHARED((N_HE_PAD, c), jnp.float32),
        ])


def _e2v_body(c, he_hbm, vi_hbm, ei_hbm, z_hbm, nodes_out,
              vbuf, ebuf, rows, nodes_sh):
    cid = lax.axis_index("c")
    sid = lax.axis_index("s")
    wid = cid * NUM_SUBCORES + sid
    pltpu.sync_copy(z_hbm, nodes_sh.at[pl.ds(sid * NODE_SLICE, NODE_SLICE)])
    pltpu.sync_copy(vi_hbm.at[wid], vbuf)
    pltpu.sync_copy(ei_hbm.at[wid], ebuf)
    plsc.subcore_barrier()

    @pl.loop(0, NCHUNK)
    def _(j):
        pltpu.sync_copy(he_hbm.at[ebuf.at[j]], rows)
        pltpu.sync_copy(rows, nodes_sh.at[vbuf.at[j]], add=True)

    plsc.subcore_barrier()
    pltpu.sync_copy(nodes_sh.at[pl.ds(sid * NODE_SLICE, NODE_SLICE)],
                    nodes_out.at[cid, pl.ds(sid * NODE_SLICE, NODE_SLICE)])


def _make_e2v(c):
    return pl.kernel(
        functools.partial(_e2v_body, c),
        out_type=jax.ShapeDtypeStruct((NUM_CORES, N_NODES_PAD, c), jnp.float32),
        mesh=_MESH,
        scratch_types=[
            pltpu.VMEM((NCHUNK, CHUNK), jnp.int32),
            pltpu.VMEM((NCHUNK, CHUNK), jnp.int32),
            pltpu.VMEM((CHUNK, c), jnp.float32),
            pltpu.VMEM_SHARED((N_NODES_PAD, c), jnp.float32),
        ])


def _dv_isqrt(dvp):
    dv = dvp[0, :N_NODES, :1] + dvp[1, :N_NODES, :1]
    return jnp.where(dv > 0, lax.rsqrt(jnp.maximum(dv, 1e-12)), 0.0)


def _theta0_body(x_ref, w_ref, b_ref, dvp_ref, shift_ref, o_ref):
    x = x_ref[...] + shift_ref[0, 0]
    h = jnp.dot(x, w_ref[...], preferred_element_type=jnp.float32,
                precision=lax.Precision.HIGHEST) + b_ref[...]
    o_ref[...] = h * _dv_isqrt(dvp_ref)


def _scale_he_body(hep_ref, dep_ref, o_ref):
    he = hep_ref[0] + hep_ref[1]
    de = dep_ref[0, :, :1] + dep_ref[1, :, :1]
    dei = jnp.where(de > 0, 1.0 / jnp.maximum(de, 1e-12), 0.0)
    o_ref[...] = he * dei


def _theta1_body(np_ref, dvp_ref, w_ref, b_ref, o_ref):
    dvis = _dv_isqrt(dvp_ref)
    sm = (np_ref[0, :N_NODES] + np_ref[1, :N_NODES]) * dvis
    h = jnp.maximum(sm, 0.0)
    r = (jnp.dot(h, w_ref[...], preferred_element_type=jnp.float32,
                 precision=lax.Precision.HIGHEST) + b_ref[...]) * dvis
    o_ref[...] = jnp.concatenate([r, jnp.zeros_like(r)], axis=1)


def _final_body(c_cls, np_ref, dvp_ref, o_ref):
    sm = ((np_ref[0, :N_NODES, :c_cls] + np_ref[1, :N_NODES, :c_cls])
          * _dv_isqrt(dvp_ref))
    o_ref[...] = jax.nn.sigmoid(sm)


def _tc_call(body, out_shape):
    return pl.pallas_call(body, out_shape=out_shape)


def kernel(X, W0, b0, W1, b1, v_idx, e_idx, num_hyperedges):
    c_in = X.shape[1]
    c_hid = W0.shape[1]
    c_cls = W1.shape[1]
    f32 = jnp.float32

    vi3 = v_idx.reshape(NW, NCHUNK, CHUNK)
    ei3 = e_idx.reshape(NW, NCHUNK, CHUNK)
    ones16 = jnp.ones((CHUNK, 16), f32)
    z16 = jnp.zeros((NODE_SLICE, 16), f32)
    z_he_h = jnp.zeros((HE_SLICE, c_hid), f32)
    z_nd_h = jnp.zeros((NODE_SLICE, c_hid), f32)
    shift = (jnp.asarray(num_hyperedges) - N_HE).astype(f32).reshape(1, 1)

    dvp, dep = _make_degrees()(vi3, ei3, ones16, z16)

    h0s = _tc_call(_theta0_body,
                   jax.ShapeDtypeStruct((N_NODES, c_hid), f32))(
        X, W0, b0.reshape(1, c_hid), dvp, shift)

    hep = _make_v2e(c_hid)(h0s, vi3, ei3, z_he_h)
    hes = _tc_call(_scale_he_body,
                   jax.ShapeDtypeStruct((N_HE_PAD, c_hid), f32))(hep, dep)
    ndp = _make_e2v(c_hid)(hes, vi3, ei3, z_nd_h)

    h1s = _tc_call(_theta1_body,
                   jax.ShapeDtypeStruct((N_NODES, 2 * c_cls), f32))(
        ndp, dvp, W1, b1.reshape(1, c_cls))

    hep2 = _make_v2e(c_hid)(h1s, vi3, ei3, z_he_h)
    hes2 = _tc_call(_scale_he_body,
                    jax.ShapeDtypeStruct((N_HE_PAD, c_hid), f32))(hep2, dep)
    ndp2 = _make_e2v(c_hid)(hes2, vi3, ei3, z_nd_h)

    out = _tc_call(functools.partial(_final_body, c_cls),
                   jax.ShapeDtypeStruct((N_NODES, c_cls), f32))(ndp2, dvp)
    return out


# speedup vs baseline: 9.2187x; 1.3149x over previous
import functools

import jax
import jax.numpy as jnp
from jax import lax
from jax.experimental import pallas as pl
from jax.experimental.pallas import tpu as pltpu
from jax.experimental.pallas import tpu_sc as plsc

N_NODES = 10000
N_INC = 320000
N_HE = 2500
N_HE_PAD = 2560
HE_SLICE = 160
N_NODES_PAD = 10112
NODE_SLICE = 632
NUM_CORES = 2
NUM_SUBCORES = 16
NW = NUM_CORES * NUM_SUBCORES
PER_TILE = N_INC // NW
DEG_CHUNK = 80
DEG_NCHUNK = PER_TILE // DEG_CHUNK

_MESH = plsc.VectorSubcoreMesh(
    core_axis_name="c", subcore_axis_name="s",
    num_cores=NUM_CORES, num_subcores=NUM_SUBCORES)


def _degrees_body(vi_hbm, ei_hbm, ones_hbm, z_hbm, dv_out, de_out,
                  vbuf, ebuf, ones_v, dv_sh, de_sh):
    cid = lax.axis_index("c")
    sid = lax.axis_index("s")
    wid = cid * NUM_SUBCORES + sid
    pltpu.sync_copy(z_hbm, dv_sh.at[pl.ds(sid * NODE_SLICE, NODE_SLICE)])
    pltpu.sync_copy(z_hbm.at[pl.ds(0, HE_SLICE)],
                    de_sh.at[pl.ds(sid * HE_SLICE, HE_SLICE)])
    pltpu.sync_copy(ones_hbm, ones_v)
    pltpu.sync_copy(vi_hbm.at[wid], vbuf)
    pltpu.sync_copy(ei_hbm.at[wid], ebuf)
    plsc.subcore_barrier()

    @pl.loop(0, DEG_NCHUNK)
    def _(j):
        pltpu.sync_copy(ones_v, dv_sh.at[vbuf.at[j]], add=True)
        pltpu.sync_copy(ones_v, de_sh.at[ebuf.at[j]], add=True)

    plsc.subcore_barrier()
    pltpu.sync_copy(dv_sh.at[pl.ds(sid * NODE_SLICE, NODE_SLICE)],
                    dv_out.at[cid, pl.ds(sid * NODE_SLICE, NODE_SLICE)])
    pltpu.sync_copy(de_sh.at[pl.ds(sid * HE_SLICE, HE_SLICE)],
                    de_out.at[cid, pl.ds(sid * HE_SLICE, HE_SLICE)])


def _make_degrees():
    return pl.kernel(
        _degrees_body,
        out_type=(jax.ShapeDtypeStruct((NUM_CORES, N_NODES_PAD, 16), jnp.float32),
                  jax.ShapeDtypeStruct((NUM_CORES, N_HE_PAD, 16), jnp.float32)),
        mesh=_MESH,
        scratch_types=[
            pltpu.VMEM((DEG_NCHUNK, DEG_CHUNK), jnp.int32),
            pltpu.VMEM((DEG_NCHUNK, DEG_CHUNK), jnp.int32),
            pltpu.VMEM((DEG_CHUNK, 16), jnp.float32),
            pltpu.VMEM_SHARED((N_NODES_PAD, 16), jnp.float32),
            pltpu.VMEM_SHARED((N_HE_PAD, 16), jnp.float32),
        ])


def _pipelined_pass(nchunk, ring, src, gbuf, sbuf, acc, rows, gsem, ssem):
    def gather(jj, b):
        pltpu.async_copy(src.at[gbuf.at[jj]], rows[b], gsem.at[b])

    def gather_wait(jj, b):
        pltpu.make_async_copy(src.at[gbuf.at[jj]], rows[b], gsem.at[b]).wait()

    def scatter(jj, b):
        pltpu.async_copy(rows[b], acc.at[sbuf.at[jj]], ssem.at[b], add=True)

    def scatter_wait(jj, b):
        pltpu.make_async_copy(rows[b], acc.at[sbuf.at[jj]], ssem.at[b]).wait()

    nmain = nchunk // ring - 1
    for b in range(ring):
        gather(b, b)

    @pl.loop(0, nmain)
    def _(j):
        for b in range(ring):
            jj = j * ring + b
            gather_wait(jj, b)
            scatter(jj, b)
            scatter_wait(jj, b)
            gather(jj + ring, b)

    base = nmain * ring
    for b in range(ring):
        gather_wait(base + b, b)
        scatter(base + b, b)
    for b in range(ring):
        scatter_wait(base + b, b)


def _smooth_body(swap, slice_rows, nchunk, ring, src_hbm, vi_hbm, ei_hbm,
                 z_hbm, out_hbm, *refs):
    vbuf, ebuf = refs[0], refs[1]
    rows = list(refs[2:2 + ring])
    gsem, ssem, acc_sh = refs[2 + ring:]
    cid = lax.axis_index("c")
    sid = lax.axis_index("s")
    wid = cid * NUM_SUBCORES + sid
    pltpu.sync_copy(z_hbm, acc_sh.at[pl.ds(sid * slice_rows, slice_rows)])
    pltpu.sync_copy(vi_hbm.at[wid], vbuf)
    pltpu.sync_copy(ei_hbm.at[wid], ebuf)
    plsc.subcore_barrier()
    gbuf, sbuf = (ebuf, vbuf) if swap else (vbuf, ebuf)
    _pipelined_pass(nchunk, ring, src_hbm, gbuf, sbuf, acc_sh, rows,
                    gsem, ssem)
    plsc.subcore_barrier()
    pltpu.sync_copy(acc_sh.at[pl.ds(sid * slice_rows, slice_rows)],
                    out_hbm.at[cid, pl.ds(sid * slice_rows, slice_rows)])


def _make_smooth(c, swap, chunk, ring):
    n_acc, slice_rows = ((N_NODES_PAD, NODE_SLICE) if swap
                         else (N_HE_PAD, HE_SLICE))
    nchunk = PER_TILE // chunk
    return pl.kernel(
        functools.partial(_smooth_body, swap, slice_rows, nchunk, ring),
        out_type=jax.ShapeDtypeStruct((NUM_CORES, n_acc, c), jnp.float32),
        mesh=_MESH,
        scratch_types=[
            pltpu.VMEM((nchunk, chunk), jnp.int32),
            pltpu.VMEM((nchunk, chunk), jnp.int32),
        ] + [pltpu.VMEM((chunk, c), jnp.float32) for _ in range(ring)] + [
            pltpu.SemaphoreType.DMA((ring,)),
            pltpu.SemaphoreType.DMA((ring,)),
            pltpu.VMEM_SHARED((n_acc, c), jnp.float32),
        ])


V2E_CHUNK, V2E_RING = 80, 5
E2V_CHUNK, E2V_RING = 80, 1


def _make_v2e(c):
    return _make_smooth(c, swap=False, chunk=V2E_CHUNK, ring=V2E_RING)


def _make_e2v(c):
    return _make_smooth(c, swap=True, chunk=E2V_CHUNK, ring=E2V_RING)


def _dv_isqrt(dvp):
    dv = dvp[0, :N_NODES, :1] + dvp[1, :N_NODES, :1]
    return jnp.where(dv > 0, lax.rsqrt(jnp.maximum(dv, 1e-12)), 0.0)


def _theta0_body(x_ref, w_ref, b_ref, dvp_ref, shift_ref, o_ref):
    x = x_ref[...] + shift_ref[0, 0]
    h = jnp.dot(x, w_ref[...], preferred_element_type=jnp.float32,
                precision=lax.Precision.HIGHEST) + b_ref[...]
    o_ref[...] = h * _dv_isqrt(dvp_ref)


def _scale_he_body(hep_ref, dep_ref, o_ref):
    he = hep_ref[0] + hep_ref[1]
    de = dep_ref[0, :, :1] + dep_ref[1, :, :1]
    dei = jnp.where(de > 0, 1.0 / jnp.maximum(de, 1e-12), 0.0)
    o_ref[...] = he * dei


def _theta1_body(np_ref, dvp_ref, w_ref, b_ref, o_ref):
    dvis = _dv_isqrt(dvp_ref)
    sm = (np_ref[0, :N_NODES] + np_ref[1, :N_NODES]) * dvis
    h = jnp.maximum(sm, 0.0)
    r = (jnp.dot(h, w_ref[...], preferred_element_type=jnp.float32,
                 precision=lax.Precision.HIGHEST) + b_ref[...]) * dvis
    o_ref[...] = jnp.concatenate([r, jnp.zeros_like(r)], axis=1)


def _final_body(c_cls, np_ref, dvp_ref, o_ref):
    sm = ((np_ref[0, :N_NODES, :c_cls] + np_ref[1, :N_NODES, :c_cls])
          * _dv_isqrt(dvp_ref))
    o_ref[...] = jax.nn.sigmoid(sm)


def _tc_call(body, out_shape):
    return pl.pallas_call(body, out_shape=out_shape)


def kernel(X, W0, b0, W1, b1, v_idx, e_idx, num_hyperedges):
    c_in = X.shape[1]
    c_hid = W0.shape[1]
    c_cls = W1.shape[1]
    f32 = jnp.float32

    vi80 = v_idx.reshape(NW, PER_TILE // V2E_CHUNK, V2E_CHUNK)
    ei80 = e_idx.reshape(NW, PER_TILE // V2E_CHUNK, V2E_CHUNK)
    ones16 = jnp.ones((DEG_CHUNK, 16), f32)
    z16 = jnp.zeros((NODE_SLICE, 16), f32)
    z_he_h = jnp.zeros((HE_SLICE, c_hid), f32)
    z_nd_h = jnp.zeros((NODE_SLICE, c_hid), f32)
    shift = (jnp.asarray(num_hyperedges) - N_HE).astype(f32).reshape(1, 1)

    dvp, dep = _make_degrees()(vi80, ei80, ones16, z16)

    h0s = _tc_call(_theta0_body,
                   jax.ShapeDtypeStruct((N_NODES, c_hid), f32))(
        X, W0, b0.reshape(1, c_hid), dvp, shift)

    hep = _make_v2e(c_hid)(h0s, vi80, ei80, z_he_h)
    hes = _tc_call(_scale_he_body,
                   jax.ShapeDtypeStruct((N_HE_PAD, c_hid), f32))(hep, dep)
    ndp = _make_e2v(c_hid)(hes, vi80, ei80, z_nd_h)

    h1s = _tc_call(_theta1_body,
                   jax.ShapeDtypeStruct((N_NODES, 2 * c_cls), f32))(
        ndp, dvp, W1, b1.reshape(1, c_cls))

    hep2 = _make_v2e(c_hid)(h1s, vi80, ei80, z_he_h)
    hes2 = _tc_call(_scale_he_body,
                    jax.ShapeDtypeStruct((N_HE_PAD, c_hid), f32))(hep2, dep)
    ndp2 = _make_e2v(c_hid)(hes2, vi80, ei80, z_nd_h)

    out = _tc_call(functools.partial(_final_body, c_cls),
                   jax.ShapeDtypeStruct((N_NODES, c_cls), f32))(ndp2, dvp)
    return out


# speedup vs baseline: 9.2285x; 1.0011x over previous
import functools

import jax
import jax.numpy as jnp
from jax import lax
from jax.experimental import pallas as pl
from jax.experimental.pallas import tpu as pltpu
from jax.experimental.pallas import tpu_sc as plsc

N_NODES = 10000
N_INC = 320000
N_HE = 2500
N_HE_PAD = 2560
HE_SLICE = 160
N_NODES_PAD = 10112
NODE_SLICE = 632
NUM_CORES = 2
NUM_SUBCORES = 16
NW = NUM_CORES * NUM_SUBCORES
PER_TILE = N_INC // NW
DEG_CHUNK = 80
DEG_NCHUNK = PER_TILE // DEG_CHUNK

_MESH = plsc.VectorSubcoreMesh(
    core_axis_name="c", subcore_axis_name="s",
    num_cores=NUM_CORES, num_subcores=NUM_SUBCORES)


def _degrees_body(vi_hbm, ei_hbm, ones_hbm, z_hbm, dv_out, de_out,
                  vbuf, ebuf, ones_v, dv_sh, de_sh):
    cid = lax.axis_index("c")
    sid = lax.axis_index("s")
    wid = cid * NUM_SUBCORES + sid
    pltpu.sync_copy(z_hbm, dv_sh.at[pl.ds(sid * NODE_SLICE, NODE_SLICE)])
    pltpu.sync_copy(z_hbm.at[pl.ds(0, HE_SLICE)],
                    de_sh.at[pl.ds(sid * HE_SLICE, HE_SLICE)])
    pltpu.sync_copy(ones_hbm, ones_v)
    pltpu.sync_copy(vi_hbm.at[wid], vbuf)
    pltpu.sync_copy(ei_hbm.at[wid], ebuf)
    plsc.subcore_barrier()

    @pl.loop(0, DEG_NCHUNK)
    def _(j):
        pltpu.sync_copy(ones_v, dv_sh.at[vbuf.at[j]], add=True)
        pltpu.sync_copy(ones_v, de_sh.at[ebuf.at[j]], add=True)

    plsc.subcore_barrier()
    pltpu.sync_copy(dv_sh.at[pl.ds(sid * NODE_SLICE, NODE_SLICE)],
                    dv_out.at[cid, pl.ds(sid * NODE_SLICE, NODE_SLICE)])
    pltpu.sync_copy(de_sh.at[pl.ds(sid * HE_SLICE, HE_SLICE)],
                    de_out.at[cid, pl.ds(sid * HE_SLICE, HE_SLICE)])


def _make_degrees():
    return pl.kernel(
        _degrees_body,
        out_type=(jax.ShapeDtypeStruct((NUM_CORES, N_NODES_PAD, 16), jnp.float32),
                  jax.ShapeDtypeStruct((NUM_CORES, N_HE_PAD, 16), jnp.float32)),
        mesh=_MESH,
        scratch_types=[
            pltpu.VMEM((DEG_NCHUNK, DEG_CHUNK), jnp.int32),
            pltpu.VMEM((DEG_NCHUNK, DEG_CHUNK), jnp.int32),
            pltpu.VMEM((DEG_CHUNK, 16), jnp.float32),
            pltpu.VMEM_SHARED((N_NODES_PAD, 16), jnp.float32),
            pltpu.VMEM_SHARED((N_HE_PAD, 16), jnp.float32),
        ])


def _pipelined_pass(nchunk, ring, src, gbuf, sbuf, acc, rows, gsem, ssem):
    def gather(jj, b):
        pltpu.async_copy(src.at[gbuf.at[jj]], rows[b], gsem[b])

    def gather_wait(jj, b):
        pltpu.make_async_copy(src.at[gbuf.at[jj]], rows[b], gsem[b]).wait()

    def scatter(jj, b):
        pltpu.async_copy(rows[b], acc.at[sbuf.at[jj]], ssem[b], add=True)

    def scatter_wait(jj, b):
        pltpu.make_async_copy(rows[b], acc.at[sbuf.at[jj]], ssem[b]).wait()

    nmain = nchunk // ring - 1
    for b in range(ring):
        gather(b, b)

    @pl.loop(0, nmain)
    def _(j):
        for b in range(ring):
            jj = j * ring + b
            gather_wait(jj, b)
            scatter(jj, b)
            scatter_wait(jj, b)
            gather(jj + ring, b)

    base = nmain * ring
    for b in range(ring):
        gather_wait(base + b, b)
        scatter(base + b, b)
    for b in range(ring):
        scatter_wait(base + b, b)


def _smooth_body(swap, slice_rows, nchunk, ring, src_hbm, vi_hbm, ei_hbm,
                 z_hbm, out_hbm, *refs):
    vbuf, ebuf = refs[0], refs[1]
    rows = list(refs[2:2 + ring])
    gsem = list(refs[2 + ring:2 + 2 * ring])
    ssem = list(refs[2 + 2 * ring:2 + 3 * ring])
    acc_sh = refs[2 + 3 * ring]
    cid = lax.axis_index("c")
    sid = lax.axis_index("s")
    wid = cid * NUM_SUBCORES + sid
    pltpu.sync_copy(z_hbm, acc_sh.at[pl.ds(sid * slice_rows, slice_rows)])
    pltpu.sync_copy(vi_hbm.at[wid], vbuf)
    pltpu.sync_copy(ei_hbm.at[wid], ebuf)
    plsc.subcore_barrier()
    gbuf, sbuf = (ebuf, vbuf) if swap else (vbuf, ebuf)
    _pipelined_pass(nchunk, ring, src_hbm, gbuf, sbuf, acc_sh, rows,
                    gsem, ssem)
    plsc.subcore_barrier()
    pltpu.sync_copy(acc_sh.at[pl.ds(sid * slice_rows, slice_rows)],
                    out_hbm.at[cid, pl.ds(sid * slice_rows, slice_rows)])


def _make_smooth(c, swap, chunk, ring):
    n_acc, slice_rows = ((N_NODES_PAD, NODE_SLICE) if swap
                         else (N_HE_PAD, HE_SLICE))
    nchunk = PER_TILE // chunk
    return pl.kernel(
        functools.partial(_smooth_body, swap, slice_rows, nchunk, ring),
        out_type=jax.ShapeDtypeStruct((NUM_CORES, n_acc, c), jnp.float32),
        mesh=_MESH,
        scratch_types=[
            pltpu.VMEM((nchunk, chunk), jnp.int32),
            pltpu.VMEM((nchunk, chunk), jnp.int32),
        ] + [pltpu.VMEM((chunk, c), jnp.float32) for _ in range(ring)]
          + [pltpu.SemaphoreType.DMA for _ in range(2 * ring)] + [
            pltpu.VMEM_SHARED((n_acc, c), jnp.float32),
        ])


V2E_CHUNK, V2E_RING = 80, 5
E2V_CHUNK, E2V_RING = 80, 1


def _make_v2e(c):
    return _make_smooth(c, swap=False, chunk=V2E_CHUNK, ring=V2E_RING)


def _make_e2v(c):
    return _make_smooth(c, swap=True, chunk=E2V_CHUNK, ring=E2V_RING)


def _dv_isqrt(dvp):
    dv = dvp[0, :N_NODES, :1] + dvp[1, :N_NODES, :1]
    return jnp.where(dv > 0, lax.rsqrt(jnp.maximum(dv, 1e-12)), 0.0)


def _theta0_body(x_ref, w_ref, b_ref, dvp_ref, shift_ref, o_ref):
    x = x_ref[...] + shift_ref[0, 0]
    h = jnp.dot(x, w_ref[...], preferred_element_type=jnp.float32,
                precision=lax.Precision.HIGHEST) + b_ref[...]
    o_ref[...] = h * _dv_isqrt(dvp_ref)


def _scale_he_body(hep_ref, dep_ref, o_ref):
    he = hep_ref[0] + hep_ref[1]
    de = dep_ref[0, :, :1] + dep_ref[1, :, :1]
    dei = jnp.where(de > 0, 1.0 / jnp.maximum(de, 1e-12), 0.0)
    o_ref[...] = he * dei


def _theta1_body(np_ref, dvp_ref, w_ref, b_ref, o_ref):
    dvis = _dv_isqrt(dvp_ref)
    sm = (np_ref[0, :N_NODES] + np_ref[1, :N_NODES]) * dvis
    h = jnp.maximum(sm, 0.0)
    r = (jnp.dot(h, w_ref[...], preferred_element_type=jnp.float32,
                 precision=lax.Precision.HIGHEST) + b_ref[...]) * dvis
    o_ref[...] = jnp.concatenate([r, jnp.zeros_like(r)], axis=1)


def _final_body(c_cls, np_ref, dvp_ref, o_ref):
    sm = ((np_ref[0, :N_NODES, :c_cls] + np_ref[1, :N_NODES, :c_cls])
          * _dv_isqrt(dvp_ref))
    o_ref[...] = jax.nn.sigmoid(sm)


def _tc_call(body, out_shape):
    return pl.pallas_call(body, out_shape=out_shape)


def kernel(X, W0, b0, W1, b1, v_idx, e_idx, num_hyperedges):
    c_in = X.shape[1]
    c_hid = W0.shape[1]
    c_cls = W1.shape[1]
    f32 = jnp.float32

    vi80 = v_idx.reshape(NW, PER_TILE // V2E_CHUNK, V2E_CHUNK)
    ei80 = e_idx.reshape(NW, PER_TILE // V2E_CHUNK, V2E_CHUNK)
    ones16 = jnp.ones((DEG_CHUNK, 16), f32)
    z16 = jnp.zeros((NODE_SLICE, 16), f32)
    z_he_h = jnp.zeros((HE_SLICE, c_hid), f32)
    z_nd_h = jnp.zeros((NODE_SLICE, c_hid), f32)
    shift = (jnp.asarray(num_hyperedges) - N_HE).astype(f32).reshape(1, 1)

    dvp, dep = _make_degrees()(vi80, ei80, ones16, z16)

    h0s = _tc_call(_theta0_body,
                   jax.ShapeDtypeStruct((N_NODES, c_hid), f32))(
        X, W0, b0.reshape(1, c_hid), dvp, shift)

    hep = _make_v2e(c_hid)(h0s, vi80, ei80, z_he_h)
    hes = _tc_call(_scale_he_body,
                   jax.ShapeDtypeStruct((N_HE_PAD, c_hid), f32))(hep, dep)
    ndp = _make_e2v(c_hid)(hes, vi80, ei80, z_nd_h)

    h1s = _tc_call(_theta1_body,
                   jax.ShapeDtypeStruct((N_NODES, 2 * c_cls), f32))(
        ndp, dvp, W1, b1.reshape(1, c_cls))

    hep2 = _make_v2e(c_hid)(h1s, vi80, ei80, z_he_h)
    hes2 = _tc_call(_scale_he_body,
                    jax.ShapeDtypeStruct((N_HE_PAD, c_hid), f32))(hep2, dep)
    ndp2 = _make_e2v(c_hid)(hes2, vi80, ei80, z_nd_h)

    out = _tc_call(functools.partial(_final_body, c_cls),
                   jax.ShapeDtypeStruct((N_NODES, c_cls), f32))(ndp2, dvp)
    return out
